# Initial kernel scaffold; baseline (speedup 1.0000x reference)
import jax
import jax.numpy as jnp
from jax.experimental import pallas as pl


def kernel(ids, weight):
    raise NotImplementedError("write your pallas kernel here")


# speedup vs baseline: 1.5476x; 1.5476x over previous
import functools
import math

import jax
import jax.numpy as jnp
from jax.experimental import pallas as pl
from jax.experimental.pallas import tpu as pltpu


def _dma_gather_kernel(ids_ref, w_hbm, o_ref, buf, sems, *, tile, nt, scale):
    c = pl.program_id(0)
    t = pl.program_id(1)
    slot = jax.lax.rem(t, 2)

    def issue(t_local, slot_idx):
        base = (c * nt + t_local) * tile
        unroll = 8

        def outer(j, carry):
            b = base + j * unroll
            for i in range(unroll):
                row = ids_ref[b + i]
                pltpu.make_async_copy(
                    w_hbm.at[row],
                    buf.at[slot_idx, j * unroll + i],
                    sems.at[slot_idx],
                ).start()
            return carry

        jax.lax.fori_loop(0, tile // unroll, outer, 0)

    @pl.when(t == 0)
    def _():
        issue(0, slot)

    @pl.when(t + 1 < nt)
    def _():
        issue(t + 1, 1 - slot)

    pltpu.make_async_copy(
        w_hbm.at[pl.ds(0, tile)], buf.at[slot], sems.at[slot]
    ).wait()

    o_ref[...] = buf[slot] * scale


def kernel(ids, weight):
    V, D = weight.shape
    orig_shape = ids.shape
    flat = ids.reshape(-1).astype(jnp.int32)
    n = flat.shape[0]
    scale = float(math.sqrt(D))

    flat = jnp.clip(flat, 0, V - 1)

    tile = 256
    cores = 2
    n_pad = ((n + cores * tile - 1) // (cores * tile)) * (cores * tile)
    if n_pad != n:
        flat = jnp.concatenate([flat, jnp.zeros((n_pad - n,), jnp.int32)])
    nt = n_pad // (cores * tile)

    emb = functools.partial(_dma_gather_kernel, tile=tile, nt=nt, scale=scale)
    out = pl.pallas_call(
        emb,
        out_shape=jax.ShapeDtypeStruct((n_pad, D), weight.dtype),
        grid_spec=pltpu.PrefetchScalarGridSpec(
            num_scalar_prefetch=1,
            grid=(cores, nt),
            in_specs=[pl.BlockSpec(memory_space=pl.ANY)],
            out_specs=pl.BlockSpec(
                (tile, D), lambda c, t, ids_smem: (c * nt + t, 0)
            ),
            scratch_shapes=[
                pltpu.VMEM((2, tile, D), weight.dtype),
                pltpu.SemaphoreType.DMA((2,)),
            ],
        ),
        compiler_params=pltpu.CompilerParams(
            dimension_semantics=("parallel", "arbitrary"),
        ),
    )(flat, weight)
    return out[:n].reshape(*orig_shape, D)
